# baseline (speedup 1.0000x reference)
import functools

import jax
import jax.numpy as jnp
from jax import lax
from jax.experimental import pallas as pl
from jax.experimental.pallas import tpu as pltpu
from jax.experimental.pallas import tpu_sc as plsc

DIM = 1024
BATCH = 32
ROWS_PER_WORKER = 8
NUM_WORKERS = BATCH // ROWS_PER_WORKER
DBLK = 128


def _sc_gather(t, table):
    info = plsc.get_sparse_core_info()
    nc = info.num_cores
    mesh = plsc.VectorSubcoreMesh(core_axis_name="c", subcore_axis_name="s")

    @functools.partial(
        pl.kernel,
        mesh=mesh,
        out_type=jax.ShapeDtypeStruct((BATCH, DIM), jnp.float32),
        scratch_types=[
            pltpu.VMEM((ROWS_PER_WORKER,), jnp.int32),
            pltpu.VMEM((ROWS_PER_WORKER, DIM), jnp.float32),
            pltpu.SemaphoreType.DMA,
        ],
    )
    def gather_kernel(t_hbm, table_hbm, out_hbm, idx_v, rows_v, sem):
        wid = lax.axis_index("s") * nc + lax.axis_index("c")

        @pl.when(wid < NUM_WORKERS)
        def _():
            base = wid * ROWS_PER_WORKER
            pltpu.sync_copy(t_hbm.at[pl.ds(base, ROWS_PER_WORKER)], idx_v)
            pltpu.async_copy(table_hbm.at[idx_v], rows_v, sem).wait()
            pltpu.sync_copy(rows_v, out_hbm.at[pl.ds(base, ROWS_PER_WORKER)])

    return gather_kernel(t, table)


def _tc_broadcast_add(rows, pos):
    def body(rows_ref, pos_ref, out_ref):
        out_ref[...] = (
            rows_ref[...][None, None, :, :] + pos_ref[...][None, :, :, None]
        )

    return pl.pallas_call(
        body,
        grid=(DIM // DBLK,),
        in_specs=[
            pl.BlockSpec((BATCH, DIM), lambda i: (0, 0)),
            pl.BlockSpec((DBLK, 1), lambda i: (i, 0)),
        ],
        out_specs=pl.BlockSpec((1, DBLK, BATCH, DIM), lambda i: (0, i, 0, 0)),
        out_shape=jax.ShapeDtypeStruct((1, DIM, BATCH, DIM), jnp.float32),
    )(rows, pos)


def kernel(t, table, pos_emb):
    t = t.astype(jnp.int32)
    rows = jnp.take(table, t, axis=0)
    pos = pos_emb.reshape(DIM, 1)
    return _tc_broadcast_add(rows, pos)


# speedup vs baseline: 1.0527x; 1.0527x over previous
import functools

import jax
import jax.numpy as jnp
from jax import lax
from jax.experimental import pallas as pl
from jax.experimental.pallas import tpu as pltpu
from jax.experimental.pallas import tpu_sc as plsc

DIM = 1024
BATCH = 32
ROWS_PER_WORKER = 8
NUM_WORKERS = BATCH // ROWS_PER_WORKER
DBLK = 32


def _sc_gather(t, table):
    info = plsc.get_sparse_core_info()
    nc = info.num_cores
    mesh = plsc.VectorSubcoreMesh(core_axis_name="c", subcore_axis_name="s")

    @functools.partial(
        pl.kernel,
        mesh=mesh,
        out_type=jax.ShapeDtypeStruct((BATCH, DIM), jnp.float32),
        scratch_types=[
            pltpu.VMEM((ROWS_PER_WORKER,), jnp.int32),
            pltpu.VMEM((ROWS_PER_WORKER, DIM), jnp.float32),
            pltpu.SemaphoreType.DMA,
        ],
    )
    def gather_kernel(t_hbm, table_hbm, out_hbm, idx_v, rows_v, sem):
        wid = lax.axis_index("s") * nc + lax.axis_index("c")

        @pl.when(wid < NUM_WORKERS)
        def _():
            base = wid * ROWS_PER_WORKER
            pltpu.sync_copy(t_hbm.at[pl.ds(base, ROWS_PER_WORKER)], idx_v)
            pltpu.async_copy(table_hbm.at[idx_v], rows_v, sem).wait()
            pltpu.sync_copy(rows_v, out_hbm.at[pl.ds(base, ROWS_PER_WORKER)])

    return gather_kernel(t, table)


def _tc_broadcast_add(rows, pos):
    def body(rows_ref, pos_ref, out_ref):
        out_ref[...] = (
            rows_ref[...][None, None, :, :] + pos_ref[...][None, :, :, None]
        )

    return pl.pallas_call(
        body,
        grid=(DIM // DBLK,),
        in_specs=[
            pl.BlockSpec((BATCH, DIM), lambda i: (0, 0)),
            pl.BlockSpec((DBLK, 1), lambda i: (i, 0)),
        ],
        out_specs=pl.BlockSpec((1, DBLK, BATCH, DIM), lambda i: (0, i, 0, 0)),
        out_shape=jax.ShapeDtypeStruct((1, DIM, BATCH, DIM), jnp.float32),
    )(rows, pos)


def kernel(t, table, pos_emb):
    t = t.astype(jnp.int32)
    rows = jnp.take(table, t, axis=0)
    pos = pos_emb.reshape(DIM, 1)
    return _tc_broadcast_add(rows, pos)
